# baseline (speedup 1.0000x reference)
import functools

import jax
import jax.numpy as jnp
from jax import lax
from jax.experimental import pallas as pl
from jax.experimental.pallas import tpu as pltpu
from jax.experimental.pallas import tpu_sc as plsc

N = 10000
E = 320000
D = 128

NC, NS = 2, 16
NW = NC * NS
EPT = E // NW
C = 128
ITERS = 80
EPTP = ITERS * C
NP = 10240
RPT = NP // NS
DEAD = NP - 1


def _sc_aggregate(x, srcp, dstp):
    mesh = plsc.VectorSubcoreMesh(core_axis_name="c", subcore_axis_name="s")

    @functools.partial(
        pl.kernel,
        out_type=jax.ShapeDtypeStruct((NC, NP, D), jnp.float32),
        mesh=mesh,
        scratch_types=[
            pltpu.VMEM_SHARED((NP, D), jnp.float32),
            pltpu.VMEM((C,), jnp.int32),
            pltpu.VMEM((C,), jnp.int32),
            pltpu.VMEM((ITERS, C), jnp.int32),
            pltpu.VMEM((C, D), jnp.float32),
            pltpu.VMEM((C, D), jnp.float32),
            pltpu.SemaphoreType.DMA,
            pltpu.SemaphoreType.DMA,
            pltpu.SemaphoreType.DMA,
            pltpu.SemaphoreType.DMA,
            pltpu.SemaphoreType.DMA,
            pltpu.SemaphoreType.DMA,
        ],
    )
    def agg_kernel(x_hbm, src_hbm, dst_hbm, parts_hbm,
                   acc, srcb0, srcb1, dst_v, rows0, rows1,
                   g0, g1, s0, s1, l0, l1):
        c = lax.axis_index("c")
        s = lax.axis_index("s")
        g = c * NS + s
        rows = (rows0, rows1)
        srcb = (srcb0, srcb1)
        gsem = (g0, g1)
        ssem = (s0, s1)
        lsem = (l0, l1)

        zv = jnp.zeros((16,), jnp.float32)

        def zstep(i, carry):
            rows0[i // 8, pl.ds((i % 8) * 16, 16)] = zv
            return carry

        lax.fori_loop(0, C * 8, zstep, 0)

        def zfan(k, carry):
            pltpu.async_copy(rows0, acc.at[pl.ds(s * RPT + k * C, C)], s0)
            return carry

        def zdrain(k, carry):
            pltpu.make_async_copy(rows0,
                                  acc.at[pl.ds(s * RPT + k * C, C)], s0).wait()
            return carry

        lax.fori_loop(0, RPT // C, zfan, 0)
        pltpu.sync_copy(dst_hbm.at[g], dst_v)
        pltpu.sync_copy(src_hbm.at[g, 0], srcb0)
        pltpu.sync_copy(src_hbm.at[g, 1], srcb1)
        lax.fori_loop(0, RPT // C, zdrain, 0)
        plsc.subcore_barrier()

        def gather(b):
            pltpu.async_copy(x_hbm.at[srcb[b]], rows[b], gsem[b])

        def scatter(j, b):
            pltpu.async_copy(rows[b], acc.at[dst_v.at[j]], ssem[b], add=True)


        def body(jj, carry):
            for b in range(2):
                j = 2 * jj + b
                pltpu.make_async_copy(x_hbm.at[srcb[b]],
                                      rows[b], gsem[b]).wait()

                @pl.when(jj < ITERS // 2 - 1)
                def _():
                    pltpu.async_copy(src_hbm.at[g, j + 2], srcb[b], lsem[b])

                scatter(j, b)
                pltpu.make_async_copy(rows[b],
                                      acc.at[dst_v.at[j]], ssem[b]).wait()

                @pl.when(jj < ITERS // 2 - 1)
                def _():
                    pltpu.make_async_copy(src_hbm.at[g, j + 2],
                                          srcb[b], lsem[b]).wait()
                    gather(b)

            return carry

        gather(0)
        gather(1)
        lax.fori_loop(0, ITERS // 2, body, 0)
        plsc.subcore_barrier()

        def wstep(k, carry):
            pltpu.sync_copy(acc.at[pl.ds(s * RPT + k * C, C)], rows0)
            pltpu.sync_copy(rows0, parts_hbm.at[c, pl.ds(s * RPT + k * C, C)])
            return carry

        lax.fori_loop(0, RPT // C, wstep, 0)
    return agg_kernel(x, srcp, dstp)


def _tc_body(x_ref, p_ref, w1a_ref, w1b_ref, b1_ref, w2_ref, b2_ref, o_ref):
    agg = p_ref[0] + p_ref[1]
    h = jnp.dot(agg, w1a_ref[...], preferred_element_type=jnp.float32)
    h += jnp.dot(x_ref[...], w1b_ref[...], preferred_element_type=jnp.float32)
    h = jnp.maximum(h + b1_ref[...], 0.0)
    o_ref[...] = (jnp.dot(h, w2_ref[...], preferred_element_type=jnp.float32)
                  + b2_ref[...])


def _tc_finish(x, parts, W1, b1, W2, b2):
    R = 1000
    grid = (N // R,)
    w1a = W1[:D]
    w1b = W1[D:]
    return pl.pallas_call(
        _tc_body,
        grid=grid,
        in_specs=[
            pl.BlockSpec((R, D), lambda i: (i, 0)),
            pl.BlockSpec((NC, R, D), lambda i: (0, i, 0)),
            pl.BlockSpec((D, D), lambda i: (0, 0)),
            pl.BlockSpec((D, D), lambda i: (0, 0)),
            pl.BlockSpec((1, D), lambda i: (0, 0)),
            pl.BlockSpec((D, D), lambda i: (0, 0)),
            pl.BlockSpec((1, D), lambda i: (0, 0)),
        ],
        out_specs=pl.BlockSpec((R, D), lambda i: (i, 0)),
        out_shape=jax.ShapeDtypeStruct((N, D), jnp.float32),
    )(x, parts, w1a, w1b, b1.reshape(1, D), W2, b2.reshape(1, D))


def kernel(x, edge_index, W1, b1, W2, b2):
    ei = edge_index.astype(jnp.int32)
    pad = EPTP - EPT
    srcp = jnp.pad(ei[0].reshape(NW, EPT), ((0, 0), (0, pad))
                   ).reshape(NW, ITERS, C)
    dstp = jnp.pad(ei[1].reshape(NW, EPT), ((0, 0), (0, pad)),
                   constant_values=DEAD).reshape(NW, ITERS, C)
    parts = _sc_aggregate(x, srcp, dstp)
    return _tc_finish(x, parts, W1, b1, W2, b2)


# speedup vs baseline: 1.0002x; 1.0002x over previous
import functools

import jax
import jax.numpy as jnp
from jax import lax
from jax.experimental import pallas as pl
from jax.experimental.pallas import tpu as pltpu
from jax.experimental.pallas import tpu_sc as plsc

N = 10000
E = 320000
D = 128

NC, NS = 2, 16
NW = NC * NS
EPT = E // NW
C = 128
ITERS = 80
EPTP = ITERS * C
NP = 10240
RPT = NP // NS
DEAD = NP - 1


def _sc_aggregate(x, srcp, dstp):
    mesh = plsc.VectorSubcoreMesh(core_axis_name="c", subcore_axis_name="s")

    @functools.partial(
        pl.kernel,
        out_type=jax.ShapeDtypeStruct((NC, NP, D), jnp.float32),
        mesh=mesh,
        scratch_types=[
            pltpu.VMEM_SHARED((NP, D), jnp.float32),
            pltpu.VMEM((C,), jnp.int32),
            pltpu.VMEM((C,), jnp.int32),
            pltpu.VMEM((ITERS, C), jnp.int32),
            pltpu.VMEM((C, D), jnp.float32),
            pltpu.VMEM((C, D), jnp.float32),
            pltpu.SemaphoreType.DMA,
            pltpu.SemaphoreType.DMA,
            pltpu.SemaphoreType.DMA,
            pltpu.SemaphoreType.DMA,
            pltpu.SemaphoreType.DMA,
            pltpu.SemaphoreType.DMA,
        ],
    )
    def agg_kernel(x_hbm, src_hbm, dst_hbm, parts_hbm,
                   acc, srcb0, srcb1, dst_v, rows0, rows1,
                   g0, g1, s0, s1, l0, l1):
        c = lax.axis_index("c")
        s = lax.axis_index("s")
        g = c * NS + s
        rows = (rows0, rows1)
        srcb = (srcb0, srcb1)
        gsem = (g0, g1)
        ssem = (s0, s1)
        lsem = (l0, l1)

        zv = jnp.zeros((16,), jnp.float32)

        def zstep(i, carry):
            rows0[i // 8, pl.ds((i % 8) * 16, 16)] = zv
            return carry

        lax.fori_loop(0, C * 8, zstep, 0)

        def zfan(k, carry):
            pltpu.async_copy(rows0, acc.at[pl.ds(s * RPT + k * C, C)], s0)
            return carry

        def zdrain(k, carry):
            pltpu.make_async_copy(rows0,
                                  acc.at[pl.ds(s * RPT + k * C, C)], s0).wait()
            return carry

        lax.fori_loop(0, RPT // C, zfan, 0)
        pltpu.sync_copy(dst_hbm.at[g], dst_v)
        pltpu.sync_copy(src_hbm.at[g, 0], srcb0)
        pltpu.sync_copy(src_hbm.at[g, 1], srcb1)
        lax.fori_loop(0, RPT // C, zdrain, 0)
        plsc.subcore_barrier()

        def gather(b):
            pltpu.async_copy(x_hbm.at[srcb[b]], rows[b], gsem[b])

        def scatter(j, b):
            pltpu.async_copy(rows[b], acc.at[dst_v.at[j]], ssem[b], add=True)


        def body(jj, carry):
            for b in range(2):
                j = 2 * jj + b
                pltpu.make_async_copy(x_hbm.at[srcb[b]],
                                      rows[b], gsem[b]).wait()

                @pl.when(jj < ITERS // 2 - 1)
                def _():
                    pltpu.async_copy(src_hbm.at[g, j + 2], srcb[b], lsem[b])

                scatter(j, b)
                pltpu.make_async_copy(rows[b],
                                      acc.at[dst_v.at[j]], ssem[b]).wait()

                @pl.when(jj < ITERS // 2 - 1)
                def _():
                    pltpu.make_async_copy(src_hbm.at[g, j + 2],
                                          srcb[b], lsem[b]).wait()
                    gather(b)

            return carry

        gather(0)
        gather(1)
        lax.fori_loop(0, ITERS // 2, body, 0)
        plsc.subcore_barrier()

        def wstep(k, carry):
            pltpu.sync_copy(acc.at[pl.ds(s * RPT + k * C, C)], rows0)
            pltpu.sync_copy(rows0, parts_hbm.at[c, pl.ds(s * RPT + k * C, C)])
            return carry

        lax.fori_loop(0, RPT // C, wstep, 0)
    return agg_kernel(x, srcp, dstp)


def _tc_body(x_ref, p_ref, w1a_ref, w1b_ref, b1_ref, w2_ref, b2_ref, o_ref):
    agg = p_ref[0] + p_ref[1]
    h = jnp.dot(agg, w1a_ref[...], preferred_element_type=jnp.float32)
    h += jnp.dot(x_ref[...], w1b_ref[...], preferred_element_type=jnp.float32)
    h = jnp.maximum(h + b1_ref[...], 0.0)
    o_ref[...] = (jnp.dot(h, w2_ref[...], preferred_element_type=jnp.float32)
                  + b2_ref[...])


def _tc_finish(x, parts, W1, b1, W2, b2):
    R = 1000
    grid = (N // R,)
    w1a = W1[:D]
    w1b = W1[D:]
    return pl.pallas_call(
        _tc_body,
        grid=grid,
        in_specs=[
            pl.BlockSpec((R, D), lambda i: (i, 0)),
            pl.BlockSpec((NC, R, D), lambda i: (0, i, 0)),
            pl.BlockSpec((D, D), lambda i: (0, 0)),
            pl.BlockSpec((D, D), lambda i: (0, 0)),
            pl.BlockSpec((1, D), lambda i: (0, 0)),
            pl.BlockSpec((D, D), lambda i: (0, 0)),
            pl.BlockSpec((1, D), lambda i: (0, 0)),
        ],
        out_specs=pl.BlockSpec((R, D), lambda i: (i, 0)),
        out_shape=jax.ShapeDtypeStruct((N, D), jnp.float32),
    )(x, parts, w1a, w1b, b1.reshape(1, D), W2, b2.reshape(1, D))


def kernel(x, edge_index, W1, b1, W2, b2):
    ei = edge_index.astype(jnp.int32)
    pad = EPTP - EPT
    srcp = jnp.pad(ei[0].reshape(NW, EPT), ((0, 0), (0, pad))
                   ).reshape(NW, ITERS, C)
    deadrows = (N + (jnp.arange(NW * pad, dtype=jnp.int32) % (NP - N))
                ).reshape(NW, pad)
    dstp = jnp.concatenate([ei[1].reshape(NW, EPT), deadrows],
                           axis=1).reshape(NW, ITERS, C)
    parts = _sc_aggregate(x, srcp, dstp)
    return _tc_finish(x, parts, W1, b1, W2, b2)


# speedup vs baseline: 1.4279x; 1.4277x over previous
import functools

import jax
import jax.numpy as jnp
from jax import lax
from jax.experimental import pallas as pl
from jax.experimental.pallas import tpu as pltpu
from jax.experimental.pallas import tpu_sc as plsc

N = 10000
E = 320000
D = 128

NC, NS = 2, 16
NW = NC * NS
EPT = E // NW
C = 80
ITERS = EPT // C
NP = 10240
RPT = NP // NS


def _sc_aggregate(x, srcp, dstp):
    mesh = plsc.VectorSubcoreMesh(core_axis_name="c", subcore_axis_name="s")

    @functools.partial(
        pl.kernel,
        out_type=jax.ShapeDtypeStruct((NC, NP, D), jnp.float32),
        mesh=mesh,
        scratch_types=[
            pltpu.VMEM_SHARED((NP, D), jnp.float32),
            pltpu.VMEM((C,), jnp.int32),
            pltpu.VMEM((ITERS, C), jnp.int32),
            pltpu.VMEM((C, D), jnp.float32),
            pltpu.SemaphoreType.DMA,
        ],
    )
    def agg_kernel(x_hbm, src_hbm, dst_hbm, parts_hbm,
                   acc, srcb, dst_v, rows0, g0):
        c = lax.axis_index("c")
        s = lax.axis_index("s")
        g = c * NS + s

        zv = jnp.zeros((16,), jnp.float32)

        def zstep(i, carry):
            rows0[i // 8, pl.ds((i % 8) * 16, 16)] = zv
            return carry

        lax.fori_loop(0, C * 8, zstep, 0)

        def zfan(k, carry):
            pltpu.async_copy(rows0, acc.at[pl.ds(s * RPT + k * C, C)], g0)
            return carry

        def zdrain(k, carry):
            pltpu.make_async_copy(rows0,
                                  acc.at[pl.ds(s * RPT + k * C, C)], g0).wait()
            return carry

        lax.fori_loop(0, RPT // C, zfan, 0)
        pltpu.sync_copy(dst_hbm.at[g], dst_v)
        lax.fori_loop(0, RPT // C, zdrain, 0)
        plsc.subcore_barrier()

        def step(j, carry):
            pltpu.sync_copy(src_hbm.at[g, j], srcb)
            pltpu.async_copy(x_hbm.at[srcb], rows0, g0).wait()
            pltpu.sync_copy(rows0, acc.at[dst_v.at[j]], add=True)
            return carry

        lax.fori_loop(0, ITERS, step, 0)
        plsc.subcore_barrier()

        def wstep(k, carry):
            pltpu.sync_copy(acc.at[pl.ds(s * RPT + k * C, C)], rows0)
            pltpu.sync_copy(rows0, parts_hbm.at[c, pl.ds(s * RPT + k * C, C)])
            return carry

        lax.fori_loop(0, RPT // C, wstep, 0)

    return agg_kernel(x, srcp, dstp)


def _tc_body(x_ref, p_ref, w1a_ref, w1b_ref, b1_ref, w2_ref, b2_ref, o_ref):
    agg = p_ref[0] + p_ref[1]
    h = jnp.dot(agg, w1a_ref[...], preferred_element_type=jnp.float32)
    h += jnp.dot(x_ref[...], w1b_ref[...], preferred_element_type=jnp.float32)
    h = jnp.maximum(h + b1_ref[...], 0.0)
    o_ref[...] = (jnp.dot(h, w2_ref[...], preferred_element_type=jnp.float32)
                  + b2_ref[...])


def _tc_finish(x, parts, W1, b1, W2, b2):
    R = 1000
    grid = (N // R,)
    w1a = W1[:D]
    w1b = W1[D:]
    return pl.pallas_call(
        _tc_body,
        grid=grid,
        in_specs=[
            pl.BlockSpec((R, D), lambda i: (i, 0)),
            pl.BlockSpec((NC, R, D), lambda i: (0, i, 0)),
            pl.BlockSpec((D, D), lambda i: (0, 0)),
            pl.BlockSpec((D, D), lambda i: (0, 0)),
            pl.BlockSpec((1, D), lambda i: (0, 0)),
            pl.BlockSpec((D, D), lambda i: (0, 0)),
            pl.BlockSpec((1, D), lambda i: (0, 0)),
        ],
        out_specs=pl.BlockSpec((R, D), lambda i: (i, 0)),
        out_shape=jax.ShapeDtypeStruct((N, D), jnp.float32),
    )(x, parts, w1a, w1b, b1.reshape(1, D), W2, b2.reshape(1, D))


def kernel(x, edge_index, W1, b1, W2, b2):
    ei = edge_index.astype(jnp.int32)
    srcp = ei[0].reshape(NW, ITERS, C)
    dstp = ei[1].reshape(NW, ITERS, C)
    parts = _sc_aggregate(x, srcp, dstp)
    return _tc_finish(x, parts, W1, b1, W2, b2)
